# baseline (speedup 1.0000x reference)
import jax
import jax.numpy as jnp
from jax.experimental import pallas as pl
from jax.experimental.pallas import tpu as pltpu

_N = 10000
_ROWS = 400


def _layer1_body(feat_ref, adj_ref, w1_ref, b1_ref, x1_ref, u_ref):
    @pl.when(pl.program_id(0) == 0)
    def _():
        u = jnp.dot(feat_ref[...], w1_ref[...],
                    preferred_element_type=jnp.float32)
        u_ref[...] = u.astype(jnp.bfloat16)

    a = adj_ref[...].astype(jnp.bfloat16)
    h = jnp.dot(a, u_ref[...], preferred_element_type=jnp.float32)
    x1_ref[...] = jnp.maximum(h + b1_ref[...], 0.0)


def _layer2_body(x1_ref, adj_ref, w2_ref, b2_ref, out_ref, v_ref):
    @pl.when(pl.program_id(0) == 0)
    def _():
        v = jnp.dot(x1_ref[...], w2_ref[...],
                    preferred_element_type=jnp.float32)
        v_ref[...] = v.astype(jnp.bfloat16)

    a = adj_ref[...].astype(jnp.bfloat16)
    h = jnp.dot(a, v_ref[...], preferred_element_type=jnp.float32)
    h = h + b2_ref[...]
    m = jnp.max(h, axis=1, keepdims=True)
    e = jnp.exp(h - m)
    s = jnp.sum(e, axis=1, keepdims=True)
    out_ref[...] = h - m - jnp.log(s)


def kernel(feature, adj, W1, b1, W2, b2):
    f_in = feature.shape[1]
    hid = W1.shape[1]
    dim = W2.shape[1]
    nsteps = _N // _ROWS
    b1r = b1.reshape(1, hid)
    b2r = b2.reshape(1, dim)

    x1 = pl.pallas_call(
        _layer1_body,
        grid=(nsteps,),
        in_specs=[
            pl.BlockSpec((_N, f_in), lambda i: (0, 0)),
            pl.BlockSpec((_ROWS, _N), lambda i: (i, 0)),
            pl.BlockSpec((f_in, hid), lambda i: (0, 0)),
            pl.BlockSpec((1, hid), lambda i: (0, 0)),
        ],
        out_specs=pl.BlockSpec((_ROWS, hid), lambda i: (i, 0)),
        out_shape=jax.ShapeDtypeStruct((_N, hid), jnp.float32),
        scratch_shapes=[pltpu.VMEM((_N, hid), jnp.bfloat16)],
    )(feature, adj, W1, b1r)

    out = pl.pallas_call(
        _layer2_body,
        grid=(nsteps,),
        in_specs=[
            pl.BlockSpec((_N, hid), lambda i: (0, 0)),
            pl.BlockSpec((_ROWS, _N), lambda i: (i, 0)),
            pl.BlockSpec((hid, dim), lambda i: (0, 0)),
            pl.BlockSpec((1, dim), lambda i: (0, 0)),
        ],
        out_specs=pl.BlockSpec((_ROWS, dim), lambda i: (i, 0)),
        out_shape=jax.ShapeDtypeStruct((_N, dim), jnp.float32),
        scratch_shapes=[pltpu.VMEM((_N, dim), jnp.bfloat16)],
    )(x1, adj, W2, b2r)

    return (x1, out)
